# baseline (speedup 1.0000x reference)
import functools

import jax
import jax.numpy as jnp
from jax import lax
from jax.experimental import pallas as pl
from jax.experimental.pallas import tpu as pltpu
from jax.experimental.pallas import tpu_sc as plsc

N_POINTS = 20000
N_KP = 2048
K = 32
H = 200
W = 200
PC_X0 = 0.0
PC_Y0 = -40.0
VOX = 0.05
STRIDE = 8.0

NC = 2
NS = 16
NW = NC * NS
KPW = N_KP // NW


def _proj_body(pxyz_ref, pf_ref, w1_ref, s_ref):
    s = jnp.dot(pxyz_ref[...], w1_ref[0:3, :], preferred_element_type=jnp.float32)
    s += jnp.dot(pf_ref[...], w1_ref[3:35, :], preferred_element_type=jnp.float32)
    s_ref[...] = s


def _project_points(points_xyz, point_features, W1p):
    return pl.pallas_call(
        _proj_body,
        out_shape=jax.ShapeDtypeStruct((N_POINTS, 128), jnp.float32),
    )(points_xyz, point_features, W1p)


_RB = 64


def _select_body(kp_ref, pts_ref, idx_ref, cor_ref, w4_ref):
    xs = pts_ref[0:1, :]
    ys = pts_ref[1:2, :]
    zs = pts_ref[2:3, :]
    kx = kp_ref[:, 0:1]
    ky = kp_ref[:, 1:2]
    kz = kp_ref[:, 2:3]
    d2 = (xs - kx) ** 2 + (ys - ky) ** 2 + (zs - kz) ** 2
    iota = lax.broadcasted_iota(jnp.int32, (_RB, N_POINTS), 1)
    cols = []
    for _ in range(K):
        am = jnp.argmin(d2, axis=1).astype(jnp.int32)[:, None]
        cols.append(am)
        d2 = jnp.where(iota == am, jnp.float32(1e30), d2)
    idx_ref[...] = jnp.concatenate(cols, axis=1)

    xg = (kx - PC_X0) / VOX / STRIDE
    yg = (ky - PC_Y0) / VOX / STRIDE
    x0f = jnp.floor(xg)
    y0f = jnp.floor(yg)
    x0 = x0f.astype(jnp.int32)
    y0 = y0f.astype(jnp.int32)
    x0c = jnp.clip(x0, 0, W - 1)
    x1c = jnp.clip(x0 + 1, 0, W - 1)
    y0c = jnp.clip(y0, 0, H - 1)
    y1c = jnp.clip(y0 + 1, 0, H - 1)
    fa = y0c * W + x0c
    fb = y1c * W + x0c
    fc = y0c * W + x1c
    fd = y1c * W + x1c
    cor_ref[...] = jnp.concatenate([fa, fb, fc, fd], axis=1)
    x1f = x0f + 1.0
    y1f = y0f + 1.0
    wa = (x1f - xg) * (y1f - yg)
    wb = (x1f - xg) * (yg - y0f)
    wc = (xg - x0f) * (y1f - yg)
    wd = (xg - x0f) * (yg - y0f)
    w4_ref[...] = jnp.concatenate([wa, wb, wc, wd], axis=1)


def _select_topk(keypoints, points_t):
    return pl.pallas_call(
        _select_body,
        grid=(N_KP // _RB,),
        in_specs=[
            pl.BlockSpec((_RB, 3), lambda i: (i, 0)),
            pl.BlockSpec((3, N_POINTS), lambda i: (0, 0)),
        ],
        out_specs=[
            pl.BlockSpec((_RB, K), lambda i: (i, 0)),
            pl.BlockSpec((_RB, 4), lambda i: (i, 0)),
            pl.BlockSpec((_RB, 4), lambda i: (i, 0)),
        ],
        out_shape=[
            jax.ShapeDtypeStruct((N_KP, K), jnp.int32),
            jax.ShapeDtypeStruct((N_KP, 4), jnp.int32),
            jax.ShapeDtypeStruct((N_KP, 4), jnp.float32),
        ],
    )(keypoints, points_t)


def _sc_body(s_hbm, idx_hbm, cor_hbm, bev_hbm, maxg_hbm, bev4_hbm,
             idx_v, cor_v, srows_v, mg8_v, bev_v, sem):
    wid = lax.axis_index("s") * NC + lax.axis_index("c")
    pltpu.sync_copy(idx_hbm.at[pl.ds(wid * (KPW * K), KPW * K)], idx_v)
    pltpu.sync_copy(cor_hbm.at[pl.ds(wid * (KPW * 4), KPW * 4)], cor_v)

    for j in range(2):
        pltpu.async_copy(bev_hbm.at[cor_v.at[pl.ds(j * 128, 128)]],
                         bev_v.at[pl.ds(j * 128, 128)], sem).wait()
    pltpu.sync_copy(bev_v, bev4_hbm.at[pl.ds(wid * (KPW * 4), KPW * 4)])

    def phase(p, carry):
        for j in range(2):
            pltpu.async_copy(
                s_hbm.at[idx_v.at[pl.ds(p * 256 + j * 128, 128)]],
                srows_v.at[pl.ds(j * 128, 128)], sem).wait()
        for i in range(8):
            for c in range(4):
                acc = srows_v[i * K, c * 16:(c + 1) * 16]
                for k in range(1, K):
                    acc = jnp.maximum(acc, srows_v[i * K + k, c * 16:(c + 1) * 16])
                mg8_v[i, c * 16:(c + 1) * 16] = acc
        pltpu.sync_copy(mg8_v, maxg_hbm.at[pl.ds(wid * KPW + p * 8, 8)])
        return carry

    lax.fori_loop(0, KPW // 8, phase, 0)


def _sc_gather(s_pad, idx_flat, cor_flat, bev_im):
    mesh = plsc.VectorSubcoreMesh(core_axis_name="c", subcore_axis_name="s")
    f = functools.partial(
        pl.kernel,
        out_type=[
            jax.ShapeDtypeStruct((N_KP, 64), jnp.float32),
            jax.ShapeDtypeStruct((N_KP * 4, 256), jnp.float32),
        ],
        mesh=mesh,
        scratch_types=[
            pltpu.VMEM((KPW * K,), jnp.int32),
            pltpu.VMEM((KPW * 4,), jnp.int32),
            pltpu.VMEM((256, 128), jnp.float32),
            pltpu.VMEM((8, 64), jnp.float32),
            pltpu.VMEM((KPW * 4, 256), jnp.float32),
            pltpu.SemaphoreType.DMA,
        ],
    )(_sc_body)
    return f(s_pad, idx_flat, cor_flat, bev_im)


_FB = 128


def _fuse_body(maxg_ref, kp_ref, bev4_ref, w4_ref, w1_ref, wf_ref,
               g_ref, b_ref, o_ref):
    c = jnp.dot(kp_ref[...], w1_ref[0:3, :], preferred_element_type=jnp.float32)
    g = jnp.maximum(maxg_ref[...] - c, 0.0)
    bev = w4_ref[:, 0:1] * bev4_ref[:, 0:256]
    for j in range(1, 4):
        bev += w4_ref[:, j:j + 1] * bev4_ref[:, j * 256:(j + 1) * 256]
    out = jnp.dot(g, wf_ref[0:64, :], preferred_element_type=jnp.float32)
    out += jnp.dot(bev, wf_ref[64:320, :], preferred_element_type=jnp.float32)
    o_ref[...] = jnp.maximum(out * g_ref[...] + b_ref[...], 0.0)


def _fuse(maxg, keypoints, bev4, w4, W1, Wf, gamma2, beta2):
    return pl.pallas_call(
        _fuse_body,
        grid=(N_KP // _FB,),
        in_specs=[
            pl.BlockSpec((_FB, 64), lambda i: (i, 0)),
            pl.BlockSpec((_FB, 3), lambda i: (i, 0)),
            pl.BlockSpec((_FB, 1024), lambda i: (i, 0)),
            pl.BlockSpec((_FB, 4), lambda i: (i, 0)),
            pl.BlockSpec((35, 64), lambda i: (0, 0)),
            pl.BlockSpec((320, 128), lambda i: (0, 0)),
            pl.BlockSpec((1, 128), lambda i: (0, 0)),
            pl.BlockSpec((1, 128), lambda i: (0, 0)),
        ],
        out_specs=pl.BlockSpec((_FB, 128), lambda i: (i, 0)),
        out_shape=jax.ShapeDtypeStruct((N_KP, 128), jnp.float32),
    )(maxg, keypoints, bev4, w4, W1, Wf, gamma2, beta2)


def kernel(points_xyz, point_features, keypoints, bev_features, W1, Wf,
           gamma, beta):
    W1p = jnp.pad(W1, ((0, 0), (0, 64)))
    s_pad = _project_points(points_xyz, point_features, W1p)
    idx, cor, w4 = _select_topk(keypoints, points_xyz.T)
    bev_im = jnp.transpose(bev_features[0], (1, 2, 0)).reshape(H * W, 256)
    maxg, bev4 = _sc_gather(s_pad, idx.reshape(-1), cor.reshape(-1), bev_im)
    return _fuse(maxg, keypoints, bev4.reshape(N_KP, 4 * 256), w4, W1, Wf,
                 gamma.reshape(1, 128), beta.reshape(1, 128))


# speedup vs baseline: 1.1592x; 1.1592x over previous
import functools

import jax
import jax.numpy as jnp
from jax import lax
from jax.experimental import pallas as pl
from jax.experimental.pallas import tpu as pltpu
from jax.experimental.pallas import tpu_sc as plsc

N_POINTS = 20000
N_KP = 2048
K = 32
H = 200
W = 200
PC_X0 = 0.0
PC_Y0 = -40.0
VOX = 0.05
STRIDE = 8.0

NC = 2
NS = 16
NW = NC * NS
KPW = N_KP // NW


def _proj_body(pxyz_ref, pf_ref, w1_ref, s_ref):
    s = jnp.dot(pxyz_ref[...], w1_ref[0:3, :], preferred_element_type=jnp.float32)
    s += jnp.dot(pf_ref[...], w1_ref[3:35, :], preferred_element_type=jnp.float32)
    s_ref[...] = s


def _project_points(points_xyz, point_features, W1p):
    return pl.pallas_call(
        _proj_body,
        out_shape=jax.ShapeDtypeStruct((N_POINTS, 128), jnp.float32),
    )(points_xyz, point_features, W1p)


_RB = 32


def _select_body(kp_ref, pts_ref, idx_ref, cor_ref, w4_ref):
    xs = pts_ref[0:1, :]
    ys = pts_ref[1:2, :]
    zs = pts_ref[2:3, :]
    kx = kp_ref[:, 0:1]
    ky = kp_ref[:, 1:2]
    kz = kp_ref[:, 2:3]
    d2 = (xs - kx) ** 2 + (ys - ky) ** 2 + (zs - kz) ** 2
    iota = lax.broadcasted_iota(jnp.int32, (_RB, N_POINTS), 1)
    cols = []
    for _ in range(K):
        am = jnp.argmin(d2, axis=1).astype(jnp.int32)[:, None]
        cols.append(am)
        d2 = jnp.where(iota == am, jnp.float32(1e30), d2)
    idx_ref[...] = jnp.concatenate(cols, axis=1)

    xg = (kx - PC_X0) / VOX / STRIDE
    yg = (ky - PC_Y0) / VOX / STRIDE
    x0f = jnp.floor(xg)
    y0f = jnp.floor(yg)
    x0 = x0f.astype(jnp.int32)
    y0 = y0f.astype(jnp.int32)
    x0c = jnp.clip(x0, 0, W - 1)
    x1c = jnp.clip(x0 + 1, 0, W - 1)
    y0c = jnp.clip(y0, 0, H - 1)
    y1c = jnp.clip(y0 + 1, 0, H - 1)
    fa = y0c * W + x0c
    fb = y1c * W + x0c
    fc = y0c * W + x1c
    fd = y1c * W + x1c
    cor_ref[...] = jnp.concatenate([fa, fb, fc, fd], axis=1)
    x1f = x0f + 1.0
    y1f = y0f + 1.0
    wa = (x1f - xg) * (y1f - yg)
    wb = (x1f - xg) * (yg - y0f)
    wc = (xg - x0f) * (y1f - yg)
    wd = (xg - x0f) * (yg - y0f)
    w4_ref[...] = jnp.concatenate([wa, wb, wc, wd], axis=1)


def _select_topk(keypoints, points_t):
    return pl.pallas_call(
        _select_body,
        grid=(N_KP // _RB,),
        in_specs=[
            pl.BlockSpec((_RB, 3), lambda i: (i, 0)),
            pl.BlockSpec((3, N_POINTS), lambda i: (0, 0)),
        ],
        out_specs=[
            pl.BlockSpec((_RB, K), lambda i: (i, 0)),
            pl.BlockSpec((_RB, 4), lambda i: (i, 0)),
            pl.BlockSpec((_RB, 4), lambda i: (i, 0)),
        ],
        out_shape=[
            jax.ShapeDtypeStruct((N_KP, K), jnp.int32),
            jax.ShapeDtypeStruct((N_KP, 4), jnp.int32),
            jax.ShapeDtypeStruct((N_KP, 4), jnp.float32),
        ],
    )(keypoints, points_t)


def _sc_body(s_hbm, idx_hbm, cor_hbm, bev_hbm, maxg_hbm, bev4_hbm,
             idx_v, cor_v, srows_v, mg8_v, bev_v, sem):
    wid = lax.axis_index("s") * NC + lax.axis_index("c")
    pltpu.sync_copy(idx_hbm.at[pl.ds(wid * (KPW * K), KPW * K)], idx_v)
    pltpu.sync_copy(cor_hbm.at[pl.ds(wid * (KPW * 4), KPW * 4)], cor_v)

    for j in range(2):
        pltpu.async_copy(bev_hbm.at[cor_v.at[pl.ds(j * 128, 128)]],
                         bev_v.at[pl.ds(j * 128, 128)], sem).wait()
    pltpu.sync_copy(bev_v, bev4_hbm.at[pl.ds(wid * (KPW * 4), KPW * 4)])

    def phase(p, carry):
        for j in range(2):
            pltpu.async_copy(
                s_hbm.at[idx_v.at[pl.ds(p * 256 + j * 128, 128)]],
                srows_v.at[pl.ds(j * 128, 128)], sem).wait()
        for i in range(8):
            for c in range(4):
                acc = srows_v[i * K, c * 16:(c + 1) * 16]
                for k in range(1, K):
                    acc = jnp.maximum(acc, srows_v[i * K + k, c * 16:(c + 1) * 16])
                mg8_v[i, c * 16:(c + 1) * 16] = acc
        pltpu.sync_copy(mg8_v, maxg_hbm.at[pl.ds(wid * KPW + p * 8, 8)])
        return carry

    lax.fori_loop(0, KPW // 8, phase, 0)


def _sc_gather(s_pad, idx_flat, cor_flat, bev_im):
    mesh = plsc.VectorSubcoreMesh(core_axis_name="c", subcore_axis_name="s")
    f = functools.partial(
        pl.kernel,
        out_type=[
            jax.ShapeDtypeStruct((N_KP, 64), jnp.float32),
            jax.ShapeDtypeStruct((N_KP * 4, 256), jnp.float32),
        ],
        mesh=mesh,
        scratch_types=[
            pltpu.VMEM((KPW * K,), jnp.int32),
            pltpu.VMEM((KPW * 4,), jnp.int32),
            pltpu.VMEM((256, 128), jnp.float32),
            pltpu.VMEM((8, 64), jnp.float32),
            pltpu.VMEM((KPW * 4, 256), jnp.float32),
            pltpu.SemaphoreType.DMA,
        ],
    )(_sc_body)
    return f(s_pad, idx_flat, cor_flat, bev_im)


_FB = 128


def _fuse_body(maxg_ref, kp_ref, bev4_ref, w4_ref, w1_ref, wf_ref,
               g_ref, b_ref, o_ref):
    c = jnp.dot(kp_ref[...], w1_ref[0:3, :], preferred_element_type=jnp.float32)
    g = jnp.maximum(maxg_ref[...] - c, 0.0)
    bev = w4_ref[:, 0:1] * bev4_ref[:, 0:256]
    for j in range(1, 4):
        bev += w4_ref[:, j:j + 1] * bev4_ref[:, j * 256:(j + 1) * 256]
    out = jnp.dot(g, wf_ref[0:64, :], preferred_element_type=jnp.float32)
    out += jnp.dot(bev, wf_ref[64:320, :], preferred_element_type=jnp.float32)
    o_ref[...] = jnp.maximum(out * g_ref[...] + b_ref[...], 0.0)


def _fuse(maxg, keypoints, bev4, w4, W1, Wf, gamma2, beta2):
    return pl.pallas_call(
        _fuse_body,
        grid=(N_KP // _FB,),
        in_specs=[
            pl.BlockSpec((_FB, 64), lambda i: (i, 0)),
            pl.BlockSpec((_FB, 3), lambda i: (i, 0)),
            pl.BlockSpec((_FB, 1024), lambda i: (i, 0)),
            pl.BlockSpec((_FB, 4), lambda i: (i, 0)),
            pl.BlockSpec((35, 64), lambda i: (0, 0)),
            pl.BlockSpec((320, 128), lambda i: (0, 0)),
            pl.BlockSpec((1, 128), lambda i: (0, 0)),
            pl.BlockSpec((1, 128), lambda i: (0, 0)),
        ],
        out_specs=pl.BlockSpec((_FB, 128), lambda i: (i, 0)),
        out_shape=jax.ShapeDtypeStruct((N_KP, 128), jnp.float32),
    )(maxg, keypoints, bev4, w4, W1, Wf, gamma2, beta2)


def kernel(points_xyz, point_features, keypoints, bev_features, W1, Wf,
           gamma, beta):
    W1p = jnp.pad(W1, ((0, 0), (0, 64)))
    s_pad = _project_points(points_xyz, point_features, W1p)
    idx, cor, w4 = _select_topk(keypoints, points_xyz.T)
    bev_im = jnp.transpose(bev_features[0], (1, 2, 0)).reshape(H * W, 256)
    maxg, bev4 = _sc_gather(s_pad, idx.reshape(-1), cor.reshape(-1), bev_im)
    return _fuse(maxg, keypoints, bev4.reshape(N_KP, 4 * 256), w4, W1, Wf,
                 gamma.reshape(1, 128), beta.reshape(1, 128))
